# Initial kernel scaffold; baseline (speedup 1.0000x reference)
import jax
import jax.numpy as jnp
from jax.experimental import pallas as pl


def kernel(x, edge_index, batch, atom_emb, conv_w1, conv_b1, conv_w2, conv_b2, eps, bn_gamma, bn_beta, wh, bh, wo, bo):
    raise NotImplementedError("write your pallas kernel here")


# speedup vs baseline: 5.0128x; 5.0128x over previous
import functools

import jax
import jax.numpy as jnp
from jax import lax
from jax.experimental import pallas as pl
from jax.experimental.pallas import tpu as pltpu
from jax.experimental.pallas import tpu_sc as plsc

_N = 10000
_E = 320000
_H = 128
_F = 9
_L = 5
_G = 128

_NC = 2
_NS = 16
_NW = _NC * _NS
_EPW = _E // _NW
_K = 80
_NCHUNK = _EPW // _K

_BLK = 2000
_NB = _N // _BLK

_HIGHEST = jax.lax.Precision.HIGHEST


def _dot(a, b, dims):
    return jax.lax.dot_general(a, b, (dims, ((), ())),
                               precision=jax.lax.Precision.DEFAULT,
                               preferred_element_type=jnp.float32)


def _agg_body(h_hbm, src_hbm, dst_hbm, out_hbm, src_v, dst_v, rows_v, acc_sh,
              sem):
    c = lax.axis_index("c")
    s = lax.axis_index("s")

    def _zrow(i, carry):
        for j in range(_H // 16):
            rows_v[i, pl.ds(j * 16, 16)] = jnp.zeros((16,), jnp.float32)
        return carry

    lax.fori_loop(0, _K, _zrow, 0)
    nrow_chunks = _N // _K
    for k in range(pl.cdiv(nrow_chunks, _NS)):
        cid = s + _NS * k

        @pl.when(cid < nrow_chunks)
        def _zero_chunk():
            off = pl.multiple_of(cid * _K, 8)
            pltpu.sync_copy(rows_v, acc_sh.at[pl.ds(off, _K)])

    plsc.subcore_barrier()

    ebase = (c * _NS + s) * _EPW

    def _chunk(j, carry):
        base = pl.multiple_of(ebase + j * _K, 8)
        pltpu.sync_copy(src_hbm.at[pl.ds(base, _K)], src_v)
        pltpu.sync_copy(dst_hbm.at[pl.ds(base, _K)], dst_v)
        pltpu.async_copy(h_hbm.at[src_v], rows_v, sem).wait()
        pltpu.sync_copy(rows_v, acc_sh.at[dst_v], add=True)
        return carry

    lax.fori_loop(0, _NCHUNK, _chunk, 0)
    plsc.subcore_barrier()

    for k in range(pl.cdiv(nrow_chunks, _NS)):
        cid = s + _NS * k

        @pl.when(cid < nrow_chunks)
        def _copy_chunk():
            off = pl.multiple_of(cid * _K, 8)
            pltpu.sync_copy(acc_sh.at[pl.ds(off, _K)],
                            out_hbm.at[c, pl.ds(off, _K)])


@functools.cache
def _aggregate_fn():
    return pl.kernel(
        _agg_body,
        out_type=jax.ShapeDtypeStruct((_NC, _N, _H), jnp.float32),
        mesh=plsc.VectorSubcoreMesh(core_axis_name="c", subcore_axis_name="s"),
        scratch_types=[
            pltpu.VMEM((_K,), jnp.int32),
            pltpu.VMEM((_K,), jnp.int32),
            pltpu.VMEM((_K, _H), jnp.float32),
            pltpu.VMEM_SHARED((_N, _H), jnp.float32),
            pltpu.SemaphoreType.DMA,
        ],
    )


def _aggregate(h, src, dst):
    return _aggregate_fn()(h, src, dst)


def _row_spec(cols=_H):
    return pl.BlockSpec((_BLK, cols), lambda b: (b, 0))


def _full_spec(shape):
    return pl.BlockSpec(shape, lambda b: tuple(0 for _ in shape))


_SMEM_SPEC = pl.BlockSpec(memory_space=pltpu.SMEM)


def _atom_body(x_ref, emb_ref, out_ref):
    acc = jnp.zeros((_BLK, _H), jnp.float32)
    ids = jax.lax.broadcasted_iota(jnp.int32, (_BLK, _H), 1)
    for f in range(_F):
        onehot = (x_ref[:, f:f + 1] == ids).astype(jnp.float32)
        acc = acc + _dot(onehot, emb_ref[f], ((1,), (0,)))
    out_ref[...] = acc


def _atom_encode(x, atom_emb):
    return pl.pallas_call(
        _atom_body,
        grid=(_NB,),
        in_specs=[_row_spec(_F), _full_spec((_F, _H, _H))],
        out_specs=_row_spec(),
        out_shape=jax.ShapeDtypeStruct((_N, _H), jnp.float32),
    )(x, atom_emb)


def _mlp_body(h_ref, p_ref, w1_ref, b1_ref, w2_ref, b2_ref, eps_ref,
              z_ref, stats_ref, acc_ref, *, last):
    b = pl.program_id(0)
    h = h_ref[...]
    z = (1.0 + eps_ref[0]) * h + p_ref[0] + p_ref[1]
    z = jnp.maximum(_dot(z, w1_ref[...], ((1,), (0,))) + b1_ref[...], 0.0)
    z = _dot(z, w2_ref[...], ((1,), (0,))) + b2_ref[...]
    z_ref[...] = z
    if not last:
        @pl.when(b == 0)
        def _init():
            acc_ref[...] = jnp.zeros((8, _H), jnp.float32)

        acc_ref[0:1, :] += jnp.sum(z, axis=0, keepdims=True)
        acc_ref[1:2, :] += jnp.sum(z * z, axis=0, keepdims=True)

        @pl.when(b == _NB - 1)
        def _flush():
            stats_ref[...] = acc_ref[...]


def _mlp(h, parts, w1, b1, w2, b2, eps_i, last):
    body = functools.partial(_mlp_body, last=last)
    in_specs = [
        _row_spec(),
        pl.BlockSpec((_NC, _BLK, _H), lambda b: (0, b, 0)),
        _full_spec((_H, _H)), _full_spec((1, _H)),
        _full_spec((_H, _H)), _full_spec((1, _H)),
        _SMEM_SPEC,
    ]
    out_shape = [jax.ShapeDtypeStruct((_N, _H), jnp.float32),
                 jax.ShapeDtypeStruct((8, _H), jnp.float32)]
    out_specs = [_row_spec(), _full_spec((8, _H))]
    return pl.pallas_call(
        body,
        grid=(_NB,),
        in_specs=in_specs,
        out_specs=out_specs,
        out_shape=out_shape,
        scratch_shapes=[pltpu.VMEM((8, _H), jnp.float32)],
    )(h, parts, w1, b1, w2, b2, eps_i)


def _bn_body(z_ref, stats_ref, g_ref, bb_ref, out_ref):
    mean = stats_ref[0:1, :] / _N
    var = stats_ref[1:2, :] / _N - mean * mean
    z = z_ref[...]
    z = g_ref[...] * (z - mean) * jax.lax.rsqrt(var + 1e-5) + bb_ref[...]
    out_ref[...] = jnp.maximum(z, 0.0)


def _bn_relu(z, stats, gamma, beta):
    return pl.pallas_call(
        _bn_body,
        grid=(_NB,),
        in_specs=[_row_spec(), _full_spec((8, _H)),
                  _full_spec((1, _H)), _full_spec((1, _H))],
        out_specs=_row_spec(),
        out_shape=jax.ShapeDtypeStruct((_N, _H), jnp.float32),
    )(z, stats, gamma, beta)


def _pool_body(h_ref, batch_ref, wh_ref, bh_ref, wo_ref, bo_ref, out_ref,
               acc_ref):
    b = pl.program_id(0)

    @pl.when(b == 0)
    def _init():
        acc_ref[...] = jnp.zeros((_G, _H), jnp.float32)

    ids = jax.lax.broadcasted_iota(jnp.int32, (_BLK, _G), 1)
    onehot = (batch_ref[...] == ids).astype(jnp.float32)
    acc_ref[...] += _dot(onehot, h_ref[...], ((0,), (0,)))

    @pl.when(b == _NB - 1)
    def _head():
        pooled = acc_ref[...]
        g = jnp.maximum(_dot(pooled, wh_ref[...], ((1,), (0,)))
                        + bh_ref[...], 0.0)
        res = _dot(g, wo_ref[...], ((1,), (0,)))
        out_ref[...] = res[:, 0:1] + bo_ref[0]


def _pool(h, batch2d, wh, bh, wo_pad, bo):
    return pl.pallas_call(
        _pool_body,
        grid=(_NB,),
        in_specs=[_row_spec(), _row_spec(1), _full_spec((_H, _H)),
                  _full_spec((1, _H)), _full_spec((_H, _H)), _SMEM_SPEC],
        out_specs=pl.BlockSpec((_G, 1), lambda b: (0, 0)),
        out_shape=jax.ShapeDtypeStruct((_G, 1), jnp.float32),
        scratch_shapes=[pltpu.VMEM((_G, _H), jnp.float32)],
    )(h, batch2d, wh, bh, wo_pad, bo)


def kernel(x, edge_index, batch, atom_emb, conv_w1, conv_b1, conv_w2, conv_b2,
           eps, bn_gamma, bn_beta, wh, bh, wo, bo):
    src = edge_index[0]
    dst = edge_index[1]
    h = _atom_encode(x, atom_emb)
    for i in range(_L):
        parts = _aggregate(h, src, dst)
        last = i == _L - 1
        z, stats = _mlp(h, parts, conv_w1[i], conv_b1[i].reshape(1, _H),
                        conv_w2[i], conv_b2[i].reshape(1, _H),
                        eps[i].reshape(1), last)
        if last:
            h = z
        else:
            h = _bn_relu(z, stats, bn_gamma[i].reshape(1, _H),
                         bn_beta[i].reshape(1, _H))
    wo_pad = jnp.pad(wo, ((0, 0), (0, _H - 1)))
    return _pool(h, batch.reshape(_N, 1), wh, bh.reshape(1, _H), wo_pad,
                 bo.reshape(1))
